# baseline (speedup 1.0000x reference)
import functools

import jax
import jax.numpy as jnp
from jax import lax
from jax.experimental import pallas as pl
from jax.experimental.pallas import tpu as pltpu
from jax.experimental.pallas import tpu_sc as plsc

_NUM_CLASSES = 1000
_D = 128
_BATCH = 16384
_ALPHA = 0.99

_NC = 2
_NS = 16
_NW = _NC * _NS
_ROWS_PER_W = _BATCH // _NW
_CHUNKS = _ROWS_PER_W // 128
_ACC_ROWS = 1024
_ZROWS = _ACC_ROWS // _NS
_TAIL = _NUM_CLASSES - (_NS - 1) * _ZROWS

_mesh = plsc.VectorSubcoreMesh(core_axis_name="c", subcore_axis_name="s")


@functools.partial(
    pl.kernel,
    out_type=jax.ShapeDtypeStruct((_NC, _NUM_CLASSES, _D), jnp.float32),
    mesh=_mesh,
    scratch_types=[
        pltpu.VMEM((_CHUNKS, 128), jnp.int32),
        pltpu.VMEM((_ROWS_PER_W, _D), jnp.float32),
        pltpu.VMEM((_ZROWS // 4, _D), jnp.float32),
        pltpu.VMEM_SHARED((_ACC_ROWS, _D), jnp.float32),
        pltpu.SemaphoreType.DMA,
        pltpu.SemaphoreType.DMA,
        pltpu.SemaphoreType.DMA,
        pltpu.SemaphoreType.DMA,
        pltpu.SemaphoreType.DMA,
        pltpu.SemaphoreType.DMA,
        pltpu.SemaphoreType.DMA,
        pltpu.SemaphoreType.DMA,
        pltpu.SemaphoreType.DMA,
        pltpu.SemaphoreType.DMA,
        pltpu.SemaphoreType.DMA,
    ],
)
def _seg_sum(inputs_hbm, labels_hbm, out_hbm,
             idx_v, rows_v, zero_v, acc_sh, sem_z, sem_l, sem_s, *sem_r):
    c = lax.axis_index("c")
    s = lax.axis_index("s")
    wid = s * _NC + c
    base = wid * _ROWS_PER_W

    lab_dma = [
        pltpu.async_copy(labels_hbm.at[pl.ds(base + j * 128, 128)],
                         idx_v.at[j], sem_l)
        for j in range(_CHUNKS)
    ]
    row_dma = [
        pltpu.async_copy(inputs_hbm.at[pl.ds(base + j * 64, 64)],
                         rows_v.at[pl.ds(j * 64, 64)], sem_r[j])
        for j in range(2 * _CHUNKS)
    ]

    _ZQ = _ZROWS // 4

    def _zrow(r, carry):
        for j in range(_D // 16):
            zero_v[r, pl.ds(j * 16, 16)] = jnp.zeros((16,), jnp.float32)
        return carry

    lax.fori_loop(0, _ZQ, _zrow, 0)
    z_dma = [
        pltpu.async_copy(zero_v, acc_sh.at[pl.ds(s * _ZROWS + q * _ZQ, _ZQ)],
                         sem_z)
        for q in range(4)
    ]
    for d in z_dma:
        d.wait()
    plsc.subcore_barrier()
    for d in lab_dma:
        d.wait()

    sc_dma = []
    for j in range(_CHUNKS):
        row_dma[2 * j].wait()
        row_dma[2 * j + 1].wait()
        sc_dma.append(
            pltpu.async_copy(rows_v.at[pl.ds(j * 128, 128)],
                             acc_sh.at[idx_v.at[j]], sem_s, add=True))
    for d in sc_dma:
        d.wait()

    plsc.subcore_barrier()

    @pl.when(s < _NS - 1)
    def _():
        pltpu.sync_copy(acc_sh.at[pl.ds(s * _ZROWS, _ZROWS)],
                        out_hbm.at[c, pl.ds(s * _ZROWS, _ZROWS)])

    @pl.when(s == _NS - 1)
    def _():
        pltpu.sync_copy(acc_sh.at[pl.ds((_NS - 1) * _ZROWS, _TAIL)],
                        out_hbm.at[c, pl.ds((_NS - 1) * _ZROWS, _TAIL)])


def _abs_sum_body(rates_ref, out_ref):
    out_ref[...] = jnp.sum(jnp.abs(rates_ref[...]), keepdims=True)


_abs_sum = pl.pallas_call(
    _abs_sum_body,
    out_shape=jax.ShapeDtypeStruct((1, 1), jnp.float32),
)

_BLK = 200


def _combine_body(asum_sref, rates_ref, parts_ref, out_ref):
    is_zero = asum_sref[0] == 0.0
    rates = rates_ref[...]
    seg = parts_ref[0] + parts_ref[1]
    out_ref[...] = jnp.where(is_zero, rates + seg,
                             _ALPHA * rates + (1.0 - _ALPHA) * seg)


_combine = pl.pallas_call(
    _combine_body,
    grid_spec=pltpu.PrefetchScalarGridSpec(
        num_scalar_prefetch=1,
        grid=(_NUM_CLASSES // _BLK,),
        in_specs=[
            pl.BlockSpec((_BLK, _D), lambda i, a: (i, 0)),
            pl.BlockSpec((_NC, _BLK, _D), lambda i, a: (0, i, 0)),
        ],
        out_specs=pl.BlockSpec((_BLK, _D), lambda i, a: (i, 0)),
    ),
    out_shape=jax.ShapeDtypeStruct((_NUM_CLASSES, _D), jnp.float32),
)


def kernel(inputs, labels, rates):
    asum = _abs_sum(rates)
    parts = _seg_sum(inputs, labels.astype(jnp.int32))
    return _combine(asum.reshape(1), rates, parts)


# speedup vs baseline: 1.0637x; 1.0637x over previous
import functools

import jax
import jax.numpy as jnp
from jax import lax
from jax.experimental import pallas as pl
from jax.experimental.pallas import tpu as pltpu
from jax.experimental.pallas import tpu_sc as plsc

_NUM_CLASSES = 1000
_D = 128
_BATCH = 16384
_ALPHA = 0.99

_NC = 2
_NS = 16
_NW = _NC * _NS
_ROWS_PER_W = _BATCH // _NW
_CHUNKS = _ROWS_PER_W // 128
_ACC_ROWS = 1024
_ZROWS = _ACC_ROWS // _NS
_TAIL = _NUM_CLASSES - (_NS - 1) * _ZROWS

_mesh = plsc.VectorSubcoreMesh(core_axis_name="c", subcore_axis_name="s")


@functools.partial(
    pl.kernel,
    out_type=jax.ShapeDtypeStruct((_NC, _NUM_CLASSES, _D), jnp.float32),
    mesh=_mesh,
    scratch_types=[
        pltpu.VMEM((2 * _CHUNKS, 64), jnp.int32),
        pltpu.VMEM((_ROWS_PER_W, _D), jnp.float32),
        pltpu.VMEM((_ZROWS // 4, _D), jnp.float32),
        pltpu.VMEM_SHARED((_ACC_ROWS, _D), jnp.float32),
        pltpu.SemaphoreType.DMA,
        pltpu.SemaphoreType.DMA,
        pltpu.SemaphoreType.DMA,
        pltpu.SemaphoreType.DMA,
        pltpu.SemaphoreType.DMA,
        pltpu.SemaphoreType.DMA,
        pltpu.SemaphoreType.DMA,
        pltpu.SemaphoreType.DMA,
        pltpu.SemaphoreType.DMA,
        pltpu.SemaphoreType.DMA,
        pltpu.SemaphoreType.DMA,
    ],
)
def _seg_sum(inputs_hbm, labels_hbm, out_hbm,
             idx_v, rows_v, zero_v, acc_sh, sem_z, sem_l, sem_s, *sem_r):
    c = lax.axis_index("c")
    s = lax.axis_index("s")
    wid = s * _NC + c
    base = wid * _ROWS_PER_W

    lab_dma = [
        pltpu.async_copy(labels_hbm.at[pl.ds(base + j * 64, 64)],
                         idx_v.at[j], sem_l)
        for j in range(2 * _CHUNKS)
    ]
    row_dma = [
        pltpu.async_copy(inputs_hbm.at[pl.ds(base + j * 64, 64)],
                         rows_v.at[pl.ds(j * 64, 64)], sem_r[j])
        for j in range(2 * _CHUNKS)
    ]

    _ZQ = _ZROWS // 4

    def _zrow(r, carry):
        for j in range(_D // 16):
            zero_v[r, pl.ds(j * 16, 16)] = jnp.zeros((16,), jnp.float32)
        return carry

    lax.fori_loop(0, _ZQ, _zrow, 0)
    z_dma = [
        pltpu.async_copy(zero_v, acc_sh.at[pl.ds(s * _ZROWS + q * _ZQ, _ZQ)],
                         sem_z)
        for q in range(4)
    ]
    for d in z_dma:
        d.wait()
    plsc.subcore_barrier()
    for d in lab_dma:
        d.wait()

    sc_dma = []
    for j in range(2 * _CHUNKS):
        row_dma[j].wait()
        sc_dma.append(
            pltpu.async_copy(rows_v.at[pl.ds(j * 64, 64)],
                             acc_sh.at[idx_v.at[j]], sem_s, add=True))
    for d in sc_dma:
        d.wait()

    plsc.subcore_barrier()

    @pl.when(s < _NS - 1)
    def _():
        pltpu.sync_copy(acc_sh.at[pl.ds(s * _ZROWS, _ZROWS)],
                        out_hbm.at[c, pl.ds(s * _ZROWS, _ZROWS)])

    @pl.when(s == _NS - 1)
    def _():
        pltpu.sync_copy(acc_sh.at[pl.ds((_NS - 1) * _ZROWS, _TAIL)],
                        out_hbm.at[c, pl.ds((_NS - 1) * _ZROWS, _TAIL)])


def _combine_body(rates_ref, parts_ref, out_ref):
    rates = rates_ref[...]
    seg = parts_ref[0] + parts_ref[1]
    is_zero = jnp.sum(jnp.abs(rates)) == 0.0
    out_ref[...] = jnp.where(is_zero, rates + seg,
                             _ALPHA * rates + (1.0 - _ALPHA) * seg)


_combine = pl.pallas_call(
    _combine_body,
    out_shape=jax.ShapeDtypeStruct((_NUM_CLASSES, _D), jnp.float32),
)


def kernel(inputs, labels, rates):
    parts = _seg_sum(inputs, labels.astype(jnp.int32))
    return _combine(rates, parts)


# speedup vs baseline: 1.0643x; 1.0005x over previous
import functools

import jax
import jax.numpy as jnp
from jax import lax
from jax.experimental import pallas as pl
from jax.experimental.pallas import tpu as pltpu
from jax.experimental.pallas import tpu_sc as plsc

_NUM_CLASSES = 1000
_D = 128
_BATCH = 16384
_ALPHA = 0.99

_NC = 2
_NS = 16
_NW = _NC * _NS
_ROWS_PER_W = _BATCH // _NW
_CHUNKS = _ROWS_PER_W // 128
_ACC_ROWS = 1024
_ZROWS = _ACC_ROWS // _NS
_TAIL = _NUM_CLASSES - (_NS - 1) * _ZROWS

_mesh = plsc.VectorSubcoreMesh(core_axis_name="c", subcore_axis_name="s")


@functools.partial(
    pl.kernel,
    out_type=jax.ShapeDtypeStruct((_NC, _NUM_CLASSES, _D), jnp.float32),
    mesh=_mesh,
    scratch_types=[
        pltpu.VMEM((_CHUNKS, 128), jnp.int32),
        pltpu.VMEM((_ROWS_PER_W, _D), jnp.float32),
        pltpu.VMEM((_ZROWS // 4, _D), jnp.float32),
        pltpu.VMEM_SHARED((_ACC_ROWS, _D), jnp.float32),
        pltpu.SemaphoreType.DMA,
        pltpu.SemaphoreType.DMA,
        pltpu.SemaphoreType.DMA,
        pltpu.SemaphoreType.DMA,
        pltpu.SemaphoreType.DMA,
        pltpu.SemaphoreType.DMA,
        pltpu.SemaphoreType.DMA,
        pltpu.SemaphoreType.DMA,
        pltpu.SemaphoreType.DMA,
        pltpu.SemaphoreType.DMA,
        pltpu.SemaphoreType.DMA,
    ],
)
def _seg_sum(inputs_hbm, labels_hbm, out_hbm,
             idx_v, rows_v, zero_v, acc_sh, sem_z, sem_l, sem_s, *sem_r):
    c = lax.axis_index("c")
    s = lax.axis_index("s")
    wid = s * _NC + c
    base = wid * _ROWS_PER_W

    lab_dma = [
        pltpu.async_copy(labels_hbm.at[pl.ds(base + j * 128, 128)],
                         idx_v.at[j], sem_l)
        for j in range(_CHUNKS)
    ]
    row_dma = [
        pltpu.async_copy(inputs_hbm.at[pl.ds(base + j * 64, 64)],
                         rows_v.at[pl.ds(j * 64, 64)], sem_r[j])
        for j in range(2 * _CHUNKS)
    ]

    _ZQ = _ZROWS // 4

    def _zrow(r, carry):
        for j in range(_D // 16):
            zero_v[r, pl.ds(j * 16, 16)] = jnp.zeros((16,), jnp.float32)
        return carry

    lax.fori_loop(0, _ZQ, _zrow, 0)
    z_dma = [
        pltpu.async_copy(zero_v, acc_sh.at[pl.ds(s * _ZROWS + q * _ZQ, _ZQ)],
                         sem_z)
        for q in range(4)
    ]
    for d in z_dma:
        d.wait()
    plsc.subcore_barrier()
    for d in lab_dma:
        d.wait()

    sc_dma = []
    for j in range(_CHUNKS):
        row_dma[2 * j].wait()
        row_dma[2 * j + 1].wait()
        sc_dma.append(
            pltpu.async_copy(rows_v.at[pl.ds(j * 128, 128)],
                             acc_sh.at[idx_v.at[j]], sem_s, add=True))
    for d in sc_dma:
        d.wait()

    plsc.subcore_barrier()

    @pl.when(s < _NS - 1)
    def _():
        pltpu.sync_copy(acc_sh.at[pl.ds(s * _ZROWS, _ZROWS)],
                        out_hbm.at[c, pl.ds(s * _ZROWS, _ZROWS)])

    @pl.when(s == _NS - 1)
    def _():
        pltpu.sync_copy(acc_sh.at[pl.ds((_NS - 1) * _ZROWS, _TAIL)],
                        out_hbm.at[c, pl.ds((_NS - 1) * _ZROWS, _TAIL)])


def _combine_body(rates_ref, parts_ref, out_ref):
    rates = rates_ref[...]
    seg = parts_ref[0] + parts_ref[1]
    is_zero = jnp.sum(jnp.abs(rates)) == 0.0
    out_ref[...] = jnp.where(is_zero, rates + seg,
                             _ALPHA * rates + (1.0 - _ALPHA) * seg)


_combine = pl.pallas_call(
    _combine_body,
    out_shape=jax.ShapeDtypeStruct((_NUM_CLASSES, _D), jnp.float32),
)


def kernel(inputs, labels, rates):
    parts = _seg_sum(inputs, labels.astype(jnp.int32))
    return _combine(rates, parts)
